# baseline (speedup 1.0000x reference)
import functools

import jax
import jax.numpy as jnp
from jax import lax
from jax.experimental import pallas as pl
from jax.experimental.pallas import tpu as pltpu
from jax.experimental.pallas import tpu_sc as plsc

_GENE_VOCAB = 60697
_VALUE_VOCAB = 51
_D = 512
_B, _L = 64, 1200
_N = _B * _L
_NC, _NS = 2, 16
_NW = _NC * _NS
_PER_W = _N // _NW
_C = 80
_NCHUNK = _PER_W // _C
_NK = _NCHUNK // 2
_G = _C // 16

_mesh = plsc.VectorSubcoreMesh(core_axis_name="c", subcore_axis_name="s")


@functools.partial(
    pl.kernel,
    mesh=_mesh,
    out_type=jax.ShapeDtypeStruct((_N, _D), jnp.float32),
    compiler_params=pltpu.CompilerParams(needs_layout_passes=False),
    scratch_types=[
        pltpu.VMEM((_PER_W,), jnp.int32),
        pltpu.VMEM((_PER_W,), jnp.int32),
        pltpu.VMEM((_C, _D), jnp.float32),
        pltpu.VMEM((_C, _D), jnp.float32),
        pltpu.VMEM((_VALUE_VOCAB * _D,), jnp.float32),
        pltpu.SemaphoreType.DMA,
        pltpu.SemaphoreType.DMA,
        pltpu.SemaphoreType.DMA,
        pltpu.SemaphoreType.DMA,
    ],
)
def _sc_embed(ids_hbm, vals_hbm, gene_hbm, vtab_hbm, out_hbm,
              gidx, vidx, g0, g1, vtab_l, sg0, sg1, so0, so1):
    wid = lax.axis_index("s") * _NC + lax.axis_index("c")
    base = wid * _PER_W

    pltpu.sync_copy(vtab_hbm, vtab_l)
    pltpu.sync_copy(ids_hbm.at[pl.ds(base, _PER_W)], gidx)
    pltpu.sync_copy(vals_hbm.at[pl.ds(base, _PER_W)], vidx)

    lanes = lax.iota(jnp.int32, 16)
    rowvecs = [g * 16 + lanes for g in range(_G)]

    def issue_gather(ci, gbuf, sg):
        isl = pl.ds(pl.multiple_of(ci * _C, _C), _C)
        pltpu.async_copy(gene_hbm.at[gidx.at[isl]], gbuf, sg)

    def wait_gather(ci, gbuf, sg):
        isl = pl.ds(pl.multiple_of(ci * _C, _C), _C)
        pltpu.make_async_copy(gene_hbm.at[gidx.at[isl]], gbuf, sg).wait()

    def out_slice(ci):
        return out_hbm.at[pl.ds(pl.multiple_of(base + ci * _C, _C), _C)]

    def add_values(ci, gbuf):
        vvecs = [vidx[pl.ds(pl.multiple_of(ci * _C, 16) + g * 16, 16)]
                 for g in range(_G)]

        def col_body(cb, carry):
            for u in range(4):
                cc = cb * 4 + u
                colvec = lax.bitwise_and(cc + lanes, _D - 1)
                for g in range(_G):
                    flat = (vvecs[g] << 9) + colvec
                    val = plsc.load_gather(vtab_l, [flat])
                    plsc.addupdate_scatter(gbuf, [rowvecs[g], colvec], val)
            return carry

        lax.fori_loop(0, _D // 4, col_body, 0)

    issue_gather(0, g0, sg0)

    def round_body(k, carry):
        a = 2 * k
        b = a + 1

        @pl.when(k > 0)
        def _():
            pltpu.make_async_copy(g1, out_slice(b - 2), so1).wait()

        issue_gather(b, g1, sg1)

        wait_gather(a, g0, sg0)
        add_values(a, g0)
        pltpu.async_copy(g0, out_slice(a), so0)

        @pl.when(k < _NK - 1)
        def _():
            pltpu.make_async_copy(g0, out_slice(a), so0).wait()
            issue_gather(a + 2, g0, sg0)

        wait_gather(b, g1, sg1)
        add_values(b, g1)
        pltpu.async_copy(g1, out_slice(b), so1)
        return carry

    lax.fori_loop(0, _NK, round_body, 0)
    pltpu.make_async_copy(g0, out_slice(_NCHUNK - 2), so0).wait()
    pltpu.make_async_copy(g1, out_slice(_NCHUNK - 1), so1).wait()


def kernel(input_ids, values, gene_table, value_table):
    ids = input_ids.reshape(-1).astype(jnp.int32)
    vals = values.reshape(-1).astype(jnp.int32)
    out = _sc_embed(ids, vals, gene_table, value_table.reshape(-1))
    return out.reshape(_B, _L, _D)


# speedup vs baseline: 1.8900x; 1.8900x over previous
import functools

import jax
import jax.numpy as jnp
from jax import lax
from jax.experimental import pallas as pl
from jax.experimental.pallas import tpu as pltpu
from jax.experimental.pallas import tpu_sc as plsc

_GENE_VOCAB = 60697
_VALUE_VOCAB = 51
_VPAD = 56
_D = 512
_B, _L = 64, 1200
_N = _B * _L
_NC, _NS = 2, 16
_NW = _NC * _NS
_PER_W = _N // _NW
_C = 48
_NCHUNK = _PER_W // _C
_NK = _NCHUNK // 2

_mesh = plsc.VectorSubcoreMesh(core_axis_name="c", subcore_axis_name="s")


@functools.partial(
    pl.kernel,
    mesh=_mesh,
    out_type=jax.ShapeDtypeStruct((_N, _D), jnp.float32),
    scratch_types=[
        pltpu.VMEM((_PER_W,), jnp.int32),
        pltpu.VMEM((_PER_W,), jnp.int32),
        pltpu.VMEM((_C, _D), jnp.float32),
        pltpu.VMEM((_C, _D), jnp.float32),
        pltpu.VMEM((_C, _D), jnp.float32),
        pltpu.VMEM((_C, _D), jnp.float32),
        pltpu.HBM((_NW * _VPAD, _D), jnp.float32),
        pltpu.SemaphoreType.DMA,
        pltpu.SemaphoreType.DMA,
        pltpu.SemaphoreType.DMA,
        pltpu.SemaphoreType.DMA,
        pltpu.SemaphoreType.DMA,
        pltpu.SemaphoreType.DMA,
    ],
)
def _sc_embed(ids_hbm, vals_hbm, gene_hbm, vtab_hbm, out_hbm,
              gidx, vidx, g0, v0, g1, v1, vrep,
              sg0, sv0, sg1, sv1, so0, so1):
    wid = lax.axis_index("s") * _NC + lax.axis_index("c")
    base = wid * _PER_W

    pltpu.sync_copy(vtab_hbm.at[pl.ds(0, _C)], v0)
    pltpu.sync_copy(v0, vrep.at[pl.ds(wid * _VPAD, _C)])
    pltpu.sync_copy(vtab_hbm.at[pl.ds(_C, _VPAD - _C)], v0.at[pl.ds(0, _VPAD - _C)])
    pltpu.sync_copy(v0.at[pl.ds(0, _VPAD - _C)], vrep.at[pl.ds(wid * _VPAD + _C, _VPAD - _C)])

    pltpu.sync_copy(ids_hbm.at[pl.ds(base, _PER_W)], gidx)
    pltpu.sync_copy(vals_hbm.at[pl.ds(base, _PER_W)], vidx)

    vbase = wid * _VPAD

    def rebase(i, carry):
        sl = pl.ds(pl.multiple_of(i * 16, 16), 16)
        vidx[sl] = vidx[sl] + vbase
        return carry

    lax.fori_loop(0, _PER_W // 16, rebase, 0)

    def issue_gathers(ci, gbuf, vbuf, sg, sv):
        isl = pl.ds(pl.multiple_of(ci * _C, _C), _C)
        pltpu.async_copy(gene_hbm.at[gidx.at[isl]], gbuf, sg)
        pltpu.async_copy(vrep.at[vidx.at[isl]], vbuf, sv)

    def wait_gathers(ci, gbuf, vbuf, sg, sv):
        isl = pl.ds(pl.multiple_of(ci * _C, _C), _C)
        pltpu.make_async_copy(gene_hbm.at[gidx.at[isl]], gbuf, sg).wait()
        pltpu.make_async_copy(vrep.at[vidx.at[isl]], vbuf, sv).wait()

    def out_slice(ci):
        return out_hbm.at[pl.ds(pl.multiple_of(base + ci * _C, _C), _C)]

    def add_rows(gbuf, vbuf):
        def body(r, carry):
            for j in range(_D // 16):
                sl = pl.ds(j * 16, 16)
                gbuf[r, sl] = gbuf[r, sl] + vbuf[r, sl]
            return carry
        lax.fori_loop(0, _C, body, 0)

    issue_gathers(0, g0, v0, sg0, sv0)

    def round_body(k, carry):
        a = 2 * k
        b = a + 1

        @pl.when(k > 0)
        def _():
            pltpu.make_async_copy(g1, out_slice(b - 2), so1).wait()

        issue_gathers(b, g1, v1, sg1, sv1)

        wait_gathers(a, g0, v0, sg0, sv0)
        add_rows(g0, v0)
        pltpu.async_copy(g0, out_slice(a), so0)

        @pl.when(k < _NK - 1)
        def _():
            pltpu.make_async_copy(g0, out_slice(a), so0).wait()
            issue_gathers(a + 2, g0, v0, sg0, sv0)

        wait_gathers(b, g1, v1, sg1, sv1)
        add_rows(g1, v1)
        pltpu.async_copy(g1, out_slice(b), so1)
        return carry

    lax.fori_loop(0, _NK, round_body, 0)
    pltpu.make_async_copy(g0, out_slice(_NCHUNK - 2), so0).wait()
    pltpu.make_async_copy(g1, out_slice(_NCHUNK - 1), so1).wait()


def kernel(input_ids, values, gene_table, value_table):
    ids = input_ids.reshape(-1).astype(jnp.int32)
    vals = values.reshape(-1).astype(jnp.int32)
    vtab_padded = jnp.pad(value_table, ((0, _VPAD - _VALUE_VOCAB), (0, 0)))
    out = _sc_embed(ids, vals, gene_table, vtab_padded)
    return out.reshape(_B, _L, _D)
